# baseline (speedup 1.0000x reference)
import jax
import jax.numpy as jnp
from jax.experimental import pallas as pl
from jax.experimental.pallas import tpu as pltpu

_N_TYPES = 100000
_SEQ_LEN = 200
_DBLK = 2048


def _bow_block_kernel(in_hbm, out_ref, scratch, sem):
    j = pl.program_id(0)

    @pl.when(j == 0)
    def _():
        cp = pltpu.make_async_copy(in_hbm, scratch, sem)
        cp.start()
        cp.wait()

    tokcol = scratch[0:_SEQ_LEN, 0, :]
    val = scratch[_SEQ_LEN:2 * _SEQ_LEN, 0, :]
    mask = tokcol == (j * _DBLK).astype(jnp.float32)
    out_ref[:, :] = jnp.where(mask, val, 0.0)


def kernel(tokens, vals):
    tokcol = (
        tokens.astype(jnp.int32)[:, None, None]
        - jnp.arange(_DBLK, dtype=jnp.int32)[None, None, :]
    ).astype(jnp.float32)
    val2 = jnp.broadcast_to(vals[:, None, None], (_SEQ_LEN, 1, _DBLK))
    fused = jnp.concatenate([tokcol, val2], axis=0)
    grid = (pl.cdiv(_N_TYPES, _DBLK),)
    out = pl.pallas_call(
        _bow_block_kernel,
        grid=grid,
        in_specs=[pl.BlockSpec(memory_space=pl.ANY)],
        out_specs=pl.BlockSpec((_SEQ_LEN, None, _DBLK), lambda j: (0, 0, j)),
        out_shape=jax.ShapeDtypeStruct((_SEQ_LEN, 1, _N_TYPES), jnp.float32),
        scratch_shapes=[
            pltpu.VMEM((2 * _SEQ_LEN, 1, _DBLK), jnp.float32),
            pltpu.SemaphoreType.DMA,
        ],
    )(fused)
    return out


# speedup vs baseline: 1.0002x; 1.0002x over previous
import jax
import jax.numpy as jnp
from jax.experimental import pallas as pl

_N_TYPES = 100000
_SEQ_LEN = 200
_DBLK = 8192


def _bow_block_kernel(tokcol_ref, val_ref, out_ref):
    j = pl.program_id(0)
    mask = tokcol_ref[:, :] == j * _DBLK
    out_ref[:, :] = jnp.where(mask, val_ref[:, :], 0.0)


def kernel(tokens, vals):
    tokcol = (
        tokens.astype(jnp.int32)[:, None, None]
        - jnp.arange(_DBLK, dtype=jnp.int32)[None, None, :]
    )
    val2 = jnp.broadcast_to(vals[:, None, None], (_SEQ_LEN, 1, _DBLK))
    grid = (pl.cdiv(_N_TYPES, _DBLK),)
    out = pl.pallas_call(
        _bow_block_kernel,
        grid=grid,
        in_specs=[
            pl.BlockSpec((_SEQ_LEN, None, _DBLK), lambda j: (0, 0, 0)),
            pl.BlockSpec((_SEQ_LEN, None, _DBLK), lambda j: (0, 0, 0)),
        ],
        out_specs=pl.BlockSpec((_SEQ_LEN, None, _DBLK), lambda j: (0, 0, j)),
        out_shape=jax.ShapeDtypeStruct((_SEQ_LEN, 1, _N_TYPES), jnp.float32),
    )(tokcol, val2)
    return out


# speedup vs baseline: 1.1935x; 1.1932x over previous
import jax
import jax.numpy as jnp
from jax.experimental import pallas as pl

_N_TYPES = 100000
_SEQ_LEN = 200
_DBLK = 8192


def _bow_block_kernel(tokcol_ref, val_ref, out_ref):
    j = pl.program_id(0)
    mask = tokcol_ref[:, :] == j * _DBLK
    valb = jnp.broadcast_to(val_ref[:, 0:1], (_SEQ_LEN, _DBLK))
    out_ref[:, :] = jnp.where(mask, valb, 0.0)


def kernel(tokens, vals):
    tokcol = (
        tokens.astype(jnp.int32)[:, None, None]
        - jnp.arange(_DBLK, dtype=jnp.int32)[None, None, :]
    )
    val2 = jnp.broadcast_to(vals[:, None, None], (_SEQ_LEN, 1, 128))
    grid = (pl.cdiv(_N_TYPES, _DBLK),)
    out = pl.pallas_call(
        _bow_block_kernel,
        grid=grid,
        in_specs=[
            pl.BlockSpec((_SEQ_LEN, None, _DBLK), lambda j: (0, 0, 0)),
            pl.BlockSpec((_SEQ_LEN, None, 128), lambda j: (0, 0, 0)),
        ],
        out_specs=pl.BlockSpec((_SEQ_LEN, None, _DBLK), lambda j: (0, 0, j)),
        out_shape=jax.ShapeDtypeStruct((_SEQ_LEN, 1, _N_TYPES), jnp.float32),
    )(tokcol, val2)
    return out


# speedup vs baseline: 1.2584x; 1.0544x over previous
import jax
import jax.numpy as jnp
from jax.experimental import pallas as pl

_N_TYPES = 100000
_SEQ_LEN = 200
_DBLK = 4096


def _bow_block_kernel(tok_ref, val_ref, col_ref, out_ref):
    j = pl.program_id(0)
    tokb = jnp.broadcast_to(tok_ref[:, 0:1], (_SEQ_LEN, _DBLK))
    colb = jnp.broadcast_to(col_ref[0:1, :], (_SEQ_LEN, _DBLK))
    valb = jnp.broadcast_to(val_ref[:, 0:1], (_SEQ_LEN, _DBLK))
    mask = tokb - j * _DBLK == colb
    out_ref[:, :] = jnp.where(mask, valb, 0.0)


def kernel(tokens, vals):
    tok2 = jnp.broadcast_to(tokens.astype(jnp.int32)[:, None, None], (_SEQ_LEN, 1, 128))
    val2 = jnp.broadcast_to(vals[:, None, None], (_SEQ_LEN, 1, 128))
    col2 = jnp.arange(_DBLK, dtype=jnp.int32)[None, None, :] * jnp.ones((8, 1, 1), jnp.int32)
    grid = (pl.cdiv(_N_TYPES, _DBLK),)
    out = pl.pallas_call(
        _bow_block_kernel,
        grid=grid,
        in_specs=[
            pl.BlockSpec((_SEQ_LEN, None, 128), lambda j: (0, 0, 0)),
            pl.BlockSpec((_SEQ_LEN, None, 128), lambda j: (0, 0, 0)),
            pl.BlockSpec((8, None, _DBLK), lambda j: (0, 0, 0)),
        ],
        out_specs=pl.BlockSpec((_SEQ_LEN, None, _DBLK), lambda j: (0, 0, j)),
        out_shape=jax.ShapeDtypeStruct((_SEQ_LEN, 1, _N_TYPES), jnp.float32),
    )(tok2, val2, col2)
    return out


# speedup vs baseline: 1.4170x; 1.1260x over previous
import jax
import jax.numpy as jnp
from jax.experimental import pallas as pl

_N_TYPES = 100000
_SEQ_LEN = 200
_DBLK = 8192


def _bow_block_kernel(tok_ref, val_ref, col_ref, out_ref):
    j = pl.program_id(0)
    tokb = jnp.broadcast_to(tok_ref[:, 0:1], (_SEQ_LEN, _DBLK))
    colb = jnp.broadcast_to(col_ref[0:1, :], (_SEQ_LEN, _DBLK))
    valb = jnp.broadcast_to(val_ref[:, 0:1], (_SEQ_LEN, _DBLK))
    mask = tokb - j * _DBLK == colb
    out_ref[:, :] = jnp.where(mask, valb, 0.0)


def kernel(tokens, vals):
    tok2 = jnp.broadcast_to(tokens.astype(jnp.int32)[:, None, None], (_SEQ_LEN, 1, 128))
    val2 = jnp.broadcast_to(vals[:, None, None], (_SEQ_LEN, 1, 128))
    col2 = jnp.arange(_DBLK, dtype=jnp.int32)[None, None, :] * jnp.ones((8, 1, 1), jnp.int32)
    grid = (pl.cdiv(_N_TYPES, _DBLK),)
    out = pl.pallas_call(
        _bow_block_kernel,
        grid=grid,
        in_specs=[
            pl.BlockSpec((_SEQ_LEN, None, 128), lambda j: (0, 0, 0)),
            pl.BlockSpec((_SEQ_LEN, None, 128), lambda j: (0, 0, 0)),
            pl.BlockSpec((8, None, _DBLK), lambda j: (0, 0, 0)),
        ],
        out_specs=pl.BlockSpec((_SEQ_LEN, None, _DBLK), lambda j: (0, 0, j)),
        out_shape=jax.ShapeDtypeStruct((_SEQ_LEN, 1, _N_TYPES), jnp.float32),
    )(tok2, val2, col2)
    return out


# speedup vs baseline: 1.4342x; 1.0121x over previous
import jax
import jax.numpy as jnp
from jax.experimental import pallas as pl

_N_TYPES = 100000
_SEQ_LEN = 200
_DBLK = 8192


def _bow_block_kernel(tok_ref, val_ref, col_ref, out_ref):
    j = pl.program_id(0)
    tokb = jnp.broadcast_to(tok_ref[:, 0:1], (_SEQ_LEN, _DBLK))
    valb = jnp.broadcast_to(val_ref[:, 0:1], (_SEQ_LEN, _DBLK))
    colb = jnp.tile(col_ref[0:8, :], (_SEQ_LEN // 8, 1))
    mask = tokb - j * _DBLK == colb
    out_ref[:, :] = jnp.where(mask, valb, 0.0)


def kernel(tokens, vals):
    tok2 = jnp.broadcast_to(tokens.astype(jnp.int32)[:, None, None], (_SEQ_LEN, 1, 128))
    val2 = jnp.broadcast_to(vals[:, None, None], (_SEQ_LEN, 1, 128))
    col2 = jnp.arange(_DBLK, dtype=jnp.int32)[None, None, :] * jnp.ones((8, 1, 1), jnp.int32)
    grid = (pl.cdiv(_N_TYPES, _DBLK),)
    out = pl.pallas_call(
        _bow_block_kernel,
        grid=grid,
        in_specs=[
            pl.BlockSpec((_SEQ_LEN, None, 128), lambda j: (0, 0, 0)),
            pl.BlockSpec((_SEQ_LEN, None, 128), lambda j: (0, 0, 0)),
            pl.BlockSpec((8, None, _DBLK), lambda j: (0, 0, 0)),
        ],
        out_specs=pl.BlockSpec((_SEQ_LEN, None, _DBLK), lambda j: (0, 0, j)),
        out_shape=jax.ShapeDtypeStruct((_SEQ_LEN, 1, _N_TYPES), jnp.float32),
    )(tok2, val2, col2)
    return out
